# baseline (speedup 1.0000x reference)
import functools

import jax
import jax.numpy as jnp
import numpy as np
from jax import lax
from jax.experimental import pallas as pl
from jax.experimental.pallas import tpu as pltpu
from jax.experimental.pallas import tpu_sc as plsc

N = 10000
E = 320000
D = 128
G = 64

NC = 2
NS = 16
NW = NC * NS

CH = 128
NCH = 80
EPW = NCH * CH
EP = EPW * NW
NPT = 632
NPT_LAST = N - NPT * (NS - 1)

RB = 1000
NB = N // RB

_PERM_INV = np.empty((D,), dtype=np.int32)
for _g in range(D // 32):
    for _k in range(16):
        _PERM_INV[32 * _g + _k] = 32 * _g + 2 * _k
        _PERM_INV[32 * _g + 16 + _k] = 32 * _g + 2 * _k + 1


def _sc_agg_body(hw_hbm, zero_hbm, src_hbm, dst_hbm, out0_hbm, out1_hbm,
                 shared, src_a, dst_a, src_b, dst_b, bf_a, bf_b, rows_f,
                 gs_a, gs_b):
    c = lax.axis_index("c")
    s = lax.axis_index("s")
    @pl.when(s < NS - 1)
    def _():
        pltpu.sync_copy(zero_hbm.at[pl.ds(s * NPT, NPT)],
                        shared.at[pl.ds(s * NPT, NPT)])

    @pl.when(s == NS - 1)
    def _():
        pltpu.sync_copy(zero_hbm.at[pl.ds(NPT * (NS - 1), NPT_LAST)],
                        shared.at[pl.ds(NPT * (NS - 1), NPT_LAST)])

    plsc.subcore_barrier()

    w = c * NS + s

    def start(j, src_v, dst_v, bf_v, gs):
        base = (w * NCH + j) * CH
        pltpu.sync_copy(src_hbm.at[pl.ds(base, CH)], src_v)
        pltpu.sync_copy(dst_hbm.at[pl.ds(base, CH)], dst_v)
        pltpu.async_copy(hw_hbm.at[src_v], bf_v, gs)

    def convert(bf_v):
        def crow(r2, carry):
            for dr in range(4):
                r = r2 * 4 + dr
                for g2 in range(D // 32):
                    wv = bf_v[r, pl.ds(16 * g2, 16)]
                    ev = lax.bitcast_convert_type(wv << 16, jnp.float32)
                    od = lax.bitcast_convert_type(wv & jnp.int32(-65536),
                                                  jnp.float32)
                    rows_f[r, pl.ds(32 * g2, 16)] = ev
                    rows_f[r, pl.ds(32 * g2 + 16, 16)] = od
            return carry

        lax.fori_loop(0, CH // 4, crow, 0)

    def finish(src_v, dst_v, bf_v, gs):
        pltpu.make_async_copy(hw_hbm.at[src_v], bf_v, gs).wait()
        convert(bf_v)
        pltpu.sync_copy(rows_f, shared.at[dst_v], add=True)

    start(0, src_a, dst_a, bf_a, gs_a)

    def body(jj, carry):
        start(2 * jj + 1, src_b, dst_b, bf_b, gs_b)
        finish(src_a, dst_a, bf_a, gs_a)
        start(2 * jj + 2, src_a, dst_a, bf_a, gs_a)
        finish(src_b, dst_b, bf_b, gs_b)
        return carry

    lax.fori_loop(0, NCH // 2 - 1, body, 0)
    start(NCH - 1, src_b, dst_b, bf_b, gs_b)
    finish(src_a, dst_a, bf_a, gs_a)
    finish(src_b, dst_b, bf_b, gs_b)
    plsc.subcore_barrier()

    @pl.when(jnp.logical_and(c == 0, s < NS - 1))
    def _():
        pltpu.sync_copy(shared.at[pl.ds(s * NPT, NPT)],
                        out0_hbm.at[pl.ds(s * NPT, NPT)])

    @pl.when(jnp.logical_and(c == 0, s == NS - 1))
    def _():
        pltpu.sync_copy(shared.at[pl.ds(NPT * (NS - 1), NPT_LAST)],
                        out0_hbm.at[pl.ds(NPT * (NS - 1), NPT_LAST)])

    @pl.when(jnp.logical_and(c == 1, s < NS - 1))
    def _():
        pltpu.sync_copy(shared.at[pl.ds(s * NPT, NPT)],
                        out1_hbm.at[pl.ds(s * NPT, NPT)])

    @pl.when(jnp.logical_and(c == 1, s == NS - 1))
    def _():
        pltpu.sync_copy(shared.at[pl.ds(NPT * (NS - 1), NPT_LAST)],
                        out1_hbm.at[pl.ds(NPT * (NS - 1), NPT_LAST)])


_sc_agg = pl.kernel(
    _sc_agg_body,
    out_type=(jax.ShapeDtypeStruct((N, D), jnp.float32),
              jax.ShapeDtypeStruct((N, D), jnp.float32)),
    mesh=plsc.VectorSubcoreMesh(core_axis_name="c", subcore_axis_name="s"),
    compiler_params=pltpu.CompilerParams(use_tc_tiling_on_sc=False),
    scratch_types=[
        pltpu.VMEM_SHARED((N + 8, D), jnp.float32),
        pltpu.VMEM((CH,), jnp.int32),
        pltpu.VMEM((CH,), jnp.int32),
        pltpu.VMEM((CH,), jnp.int32),
        pltpu.VMEM((CH,), jnp.int32),
        pltpu.VMEM((CH, D // 2), jnp.int32),
        pltpu.VMEM((CH, D // 2), jnp.int32),
        pltpu.VMEM((CH, D), jnp.float32),
        pltpu.SemaphoreType.DMA,
        pltpu.SemaphoreType.DMA,
    ],
)


def _mlp_body(a0_ref, a1_ref, h_ref, wa_ref, wap_ref, ba_ref, wb_ref, bb_ref,
              g_ref, b_ref, o_ref, ob_ref, *, ln):
    agg = a0_ref[...] + a1_ref[...]
    t = (jnp.dot(h_ref[...], wa_ref[...], preferred_element_type=jnp.float32)
         + jnp.dot(agg, wap_ref[...], preferred_element_type=jnp.float32)
         + ba_ref[...])
    t = jnp.maximum(t, 0.0)
    v = jnp.dot(t, wb_ref[...], preferred_element_type=jnp.float32) + bb_ref[...]
    if ln:
        r = jnp.maximum(v, 0.0)
        mu = jnp.mean(r, axis=-1, keepdims=True)
        var = jnp.mean(r * r, axis=-1, keepdims=True) - mu * mu
        v = (r - mu) * lax.rsqrt(var + 1e-5) * g_ref[...] + b_ref[...]
    o_ref[...] = v
    ob_ref[...] = v.astype(jnp.bfloat16)


def _make_mlp(ln):
    row_spec = pl.BlockSpec((RB, D), lambda i: (i, 0))
    roww_spec = pl.BlockSpec((RB, D), lambda i: (i, 0))
    w_spec = pl.BlockSpec((D, D), lambda i: (0, 0))
    b_spec = pl.BlockSpec((1, D), lambda i: (0, 0))
    return pl.pallas_call(
        functools.partial(_mlp_body, ln=ln),
        grid=(NB,),
        in_specs=[row_spec, row_spec, row_spec,
                  w_spec, w_spec, b_spec, w_spec, b_spec, b_spec, b_spec],
        out_specs=(row_spec, roww_spec),
        out_shape=(jax.ShapeDtypeStruct((N, D), jnp.float32),
                   jax.ShapeDtypeStruct((N, D), jnp.bfloat16)),
    )


_mlp_ln = _make_mlp(True)
_mlp_plain = _make_mlp(False)


def _pool_body(emb_ref, seg_ref, wp1_ref, bp1_ref, wp2_ref, bp2_ref,
               o_ref, acc_ref, cnt_ref):
    i = pl.program_id(0)
    r = jnp.maximum(emb_ref[...], 0.0)
    sv = seg_ref[0, 0, :]
    gid = lax.broadcasted_iota(jnp.int32, (G, RB), 0)
    oh = (sv[None, :] == gid).astype(jnp.float32)
    pa = jnp.dot(oh, r, preferred_element_type=jnp.float32)
    pc = jnp.broadcast_to(jnp.sum(oh, axis=1, keepdims=True), (G, D))

    @pl.when(i == 0)
    def _():
        acc_ref[...] = pa
        cnt_ref[...] = pc

    @pl.when(i > 0)
    def _():
        acc_ref[...] += pa
        cnt_ref[...] += pc

    @pl.when(i == NB - 1)
    def _():
        pooled = acc_ref[...] / jnp.maximum(cnt_ref[...], 1.0)
        z = jnp.dot(pooled, wp1_ref[...], preferred_element_type=jnp.float32) + bp1_ref[...]
        z = jnp.dot(z, wp2_ref[...], preferred_element_type=jnp.float32) + bp2_ref[...]
        m = jnp.max(z, axis=-1, keepdims=True)
        lse = m + jnp.log(jnp.sum(jnp.exp(z - m), axis=-1, keepdims=True))
        o_ref[...] = z - lse


_pool = pl.pallas_call(
    _pool_body,
    grid=(NB,),
    in_specs=[
        pl.BlockSpec((RB, D), lambda i: (i, 0)),
        pl.BlockSpec((1, 1, RB), lambda i: (i, 0, 0)),
        pl.BlockSpec((D, D), lambda i: (0, 0)),
        pl.BlockSpec((1, D), lambda i: (0, 0)),
        pl.BlockSpec((D, D), lambda i: (0, 0)),
        pl.BlockSpec((1, D), lambda i: (0, 0)),
    ],
    out_specs=pl.BlockSpec((G, D), lambda i: (0, 0)),
    out_shape=jax.ShapeDtypeStruct((G, D), jnp.float32),
    scratch_shapes=[
        pltpu.VMEM((G, D), jnp.float32),
        pltpu.VMEM((G, D), jnp.float32),
    ],
)


def kernel(x, edge_index, batch, W0a, b0a, W0b, b0b, W1a, b1a, W1b, b1b,
           W2a, b2a, W2b, b2b, ln0_g, ln0_b, ln1_g, ln1_b, Wp1, bp1, Wp2, bp2):
    src = edge_index[0].astype(jnp.int32)
    dst = edge_index[1].astype(jnp.int32)
    pad = EP - E
    srcp = jnp.concatenate([src, jnp.zeros((pad,), jnp.int32)])
    dstp = jnp.concatenate([dst, jnp.full((pad,), N, jnp.int32)])
    segs = batch.astype(jnp.int32).reshape(NB, 1, RB)
    zero = jnp.zeros((N, D), jnp.float32)
    pinv = jnp.asarray(_PERM_INV)

    layers = [
        (W0a, b0a, W0b, b0b, ln0_g, ln0_b, True),
        (W1a, b1a, W1b, b1b, ln1_g, ln1_b, True),
        (W2a, b2a, W2b, b2b, ln1_g, ln1_b, False),
    ]
    h = x
    hb = x.astype(jnp.bfloat16)
    for Wa, ba, Wb, bb, g, b, ln in layers:
        hw = lax.bitcast_convert_type(hb.reshape(N, D // 2, 2), jnp.int32)
        a0, a1 = _sc_agg(hw, zero, srcp, dstp)
        Wap = Wa[pinv, :]
        mlp = _mlp_ln if ln else _mlp_plain
        h, hb = mlp(a0, a1, h, Wa, Wap, ba.reshape(1, D), Wb,
                    bb.reshape(1, D), g.reshape(1, D), b.reshape(1, D))
    emb = h
    logp = _pool(emb, segs, Wp1, bp1.reshape(1, D), Wp2, bp2.reshape(1, D))
    return (emb, logp)


# speedup vs baseline: 1.0087x; 1.0087x over previous
import functools

import jax
import jax.numpy as jnp
import numpy as np
from jax import lax
from jax.experimental import pallas as pl
from jax.experimental.pallas import tpu as pltpu
from jax.experimental.pallas import tpu_sc as plsc

N = 10000
E = 320000
D = 128
G = 64

NC = 2
NS = 16
NW = NC * NS

CH = 128
NCH = 80
EPW = NCH * CH
EP = EPW * NW
NPT = 632
NPT_LAST = N - NPT * (NS - 1)

RB = 1000
NB = N // RB

_PERM_INV = np.empty((D,), dtype=np.int32)
for _g in range(D // 32):
    for _k in range(16):
        _PERM_INV[32 * _g + _k] = 32 * _g + 2 * _k
        _PERM_INV[32 * _g + 16 + _k] = 32 * _g + 2 * _k + 1


def _sc_agg_body(hw_hbm, zero_hbm, src_hbm, dst_hbm, out0_hbm, out1_hbm,
                 shared, src_a, dst_a, src_b, dst_b, bf_a, bf_b, rows_f,
                 gs_a, gs_b):
    c = lax.axis_index("c")
    s = lax.axis_index("s")
    @pl.when(s < NS - 1)
    def _():
        pltpu.sync_copy(zero_hbm.at[pl.ds(s * NPT, NPT)],
                        shared.at[pl.ds(s * NPT, NPT)])

    @pl.when(s == NS - 1)
    def _():
        pltpu.sync_copy(zero_hbm.at[pl.ds(NPT * (NS - 1), NPT_LAST)],
                        shared.at[pl.ds(NPT * (NS - 1), NPT_LAST)])

    plsc.subcore_barrier()

    w = c * NS + s

    def start(j, src_v, dst_v, bf_v, gs):
        base = (w * NCH + j) * CH
        pltpu.sync_copy(src_hbm.at[pl.ds(base, CH)], src_v)
        pltpu.sync_copy(dst_hbm.at[pl.ds(base, CH)], dst_v)
        pltpu.async_copy(hw_hbm.at[src_v], bf_v, gs)

    def convert(bf_v):
        def crow(r, carry):
            for g2 in range(D // 32):
                wv = bf_v[r, pl.ds(16 * g2, 16)]
                ev = lax.bitcast_convert_type(wv << 16, jnp.float32)
                od = lax.bitcast_convert_type(wv & jnp.int32(-65536), jnp.float32)
                rows_f[r, pl.ds(32 * g2, 16)] = ev
                rows_f[r, pl.ds(32 * g2 + 16, 16)] = od
            return carry

        lax.fori_loop(0, CH, crow, 0)

    def finish(src_v, dst_v, bf_v, gs):
        pltpu.make_async_copy(hw_hbm.at[src_v], bf_v, gs).wait()
        convert(bf_v)
        pltpu.sync_copy(rows_f, shared.at[dst_v], add=True)

    start(0, src_a, dst_a, bf_a, gs_a)

    def body(jj, carry):
        start(2 * jj + 1, src_b, dst_b, bf_b, gs_b)
        finish(src_a, dst_a, bf_a, gs_a)
        start(2 * jj + 2, src_a, dst_a, bf_a, gs_a)
        finish(src_b, dst_b, bf_b, gs_b)
        return carry

    lax.fori_loop(0, NCH // 2 - 1, body, 0)
    start(NCH - 1, src_b, dst_b, bf_b, gs_b)
    finish(src_a, dst_a, bf_a, gs_a)
    finish(src_b, dst_b, bf_b, gs_b)
    plsc.subcore_barrier()

    @pl.when(jnp.logical_and(c == 0, s < NS - 1))
    def _():
        pltpu.sync_copy(shared.at[pl.ds(s * NPT, NPT)],
                        out0_hbm.at[pl.ds(s * NPT, NPT)])

    @pl.when(jnp.logical_and(c == 0, s == NS - 1))
    def _():
        pltpu.sync_copy(shared.at[pl.ds(NPT * (NS - 1), NPT_LAST)],
                        out0_hbm.at[pl.ds(NPT * (NS - 1), NPT_LAST)])

    @pl.when(jnp.logical_and(c == 1, s < NS - 1))
    def _():
        pltpu.sync_copy(shared.at[pl.ds(s * NPT, NPT)],
                        out1_hbm.at[pl.ds(s * NPT, NPT)])

    @pl.when(jnp.logical_and(c == 1, s == NS - 1))
    def _():
        pltpu.sync_copy(shared.at[pl.ds(NPT * (NS - 1), NPT_LAST)],
                        out1_hbm.at[pl.ds(NPT * (NS - 1), NPT_LAST)])


_sc_agg = pl.kernel(
    _sc_agg_body,
    out_type=(jax.ShapeDtypeStruct((N, D), jnp.float32),
              jax.ShapeDtypeStruct((N, D), jnp.float32)),
    mesh=plsc.VectorSubcoreMesh(core_axis_name="c", subcore_axis_name="s"),
    compiler_params=pltpu.CompilerParams(use_tc_tiling_on_sc=False),
    scratch_types=[
        pltpu.VMEM_SHARED((N + 8, D), jnp.float32),
        pltpu.VMEM((CH,), jnp.int32),
        pltpu.VMEM((CH,), jnp.int32),
        pltpu.VMEM((CH,), jnp.int32),
        pltpu.VMEM((CH,), jnp.int32),
        pltpu.VMEM((CH, D // 2), jnp.int32),
        pltpu.VMEM((CH, D // 2), jnp.int32),
        pltpu.VMEM((CH, D), jnp.float32),
        pltpu.SemaphoreType.DMA,
        pltpu.SemaphoreType.DMA,
    ],
)


def _mlp_body(a0_ref, a1_ref, h_ref, wa_ref, wap_ref, ba_ref, wb_ref, bb_ref,
              g_ref, b_ref, o_ref, ob_ref, *, ln):
    agg = a0_ref[...] + a1_ref[...]
    t = (jnp.dot(h_ref[...], wa_ref[...], preferred_element_type=jnp.float32)
         + jnp.dot(agg, wap_ref[...], preferred_element_type=jnp.float32)
         + ba_ref[...])
    t = jnp.maximum(t, 0.0)
    v = jnp.dot(t, wb_ref[...], preferred_element_type=jnp.float32) + bb_ref[...]
    if ln:
        r = jnp.maximum(v, 0.0)
        mu = jnp.mean(r, axis=-1, keepdims=True)
        var = jnp.mean(r * r, axis=-1, keepdims=True) - mu * mu
        v = (r - mu) * lax.rsqrt(var + 1e-5) * g_ref[...] + b_ref[...]
    o_ref[...] = v
    ob_ref[...] = v.astype(jnp.bfloat16)


def _make_mlp(ln):
    row_spec = pl.BlockSpec((RB, D), lambda i: (i, 0))
    roww_spec = pl.BlockSpec((RB, D), lambda i: (i, 0))
    w_spec = pl.BlockSpec((D, D), lambda i: (0, 0))
    b_spec = pl.BlockSpec((1, D), lambda i: (0, 0))
    return pl.pallas_call(
        functools.partial(_mlp_body, ln=ln),
        grid=(NB,),
        in_specs=[row_spec, row_spec, row_spec,
                  w_spec, w_spec, b_spec, w_spec, b_spec, b_spec, b_spec],
        out_specs=(row_spec, roww_spec),
        out_shape=(jax.ShapeDtypeStruct((N, D), jnp.float32),
                   jax.ShapeDtypeStruct((N, D), jnp.bfloat16)),
    )


_mlp_ln = _make_mlp(True)
_mlp_plain = _make_mlp(False)


def _pool_body(emb_ref, seg_ref, wp1_ref, bp1_ref, wp2_ref, bp2_ref,
               o_ref, acc_ref, cnt_ref):
    i = pl.program_id(0)
    r = jnp.maximum(emb_ref[...], 0.0)
    sv = seg_ref[0, 0, :]
    gid = lax.broadcasted_iota(jnp.int32, (G, RB), 0)
    oh = (sv[None, :] == gid).astype(jnp.float32)
    pa = jnp.dot(oh, r, preferred_element_type=jnp.float32)
    pc = jnp.broadcast_to(jnp.sum(oh, axis=1, keepdims=True), (G, D))

    @pl.when(i == 0)
    def _():
        acc_ref[...] = pa
        cnt_ref[...] = pc

    @pl.when(i > 0)
    def _():
        acc_ref[...] += pa
        cnt_ref[...] += pc

    @pl.when(i == NB - 1)
    def _():
        pooled = acc_ref[...] / jnp.maximum(cnt_ref[...], 1.0)
        z = jnp.dot(pooled, wp1_ref[...], preferred_element_type=jnp.float32) + bp1_ref[...]
        z = jnp.dot(z, wp2_ref[...], preferred_element_type=jnp.float32) + bp2_ref[...]
        m = jnp.max(z, axis=-1, keepdims=True)
        lse = m + jnp.log(jnp.sum(jnp.exp(z - m), axis=-1, keepdims=True))
        o_ref[...] = z - lse


_pool = pl.pallas_call(
    _pool_body,
    grid=(NB,),
    in_specs=[
        pl.BlockSpec((RB, D), lambda i: (i, 0)),
        pl.BlockSpec((1, 1, RB), lambda i: (i, 0, 0)),
        pl.BlockSpec((D, D), lambda i: (0, 0)),
        pl.BlockSpec((1, D), lambda i: (0, 0)),
        pl.BlockSpec((D, D), lambda i: (0, 0)),
        pl.BlockSpec((1, D), lambda i: (0, 0)),
    ],
    out_specs=pl.BlockSpec((G, D), lambda i: (0, 0)),
    out_shape=jax.ShapeDtypeStruct((G, D), jnp.float32),
    scratch_shapes=[
        pltpu.VMEM((G, D), jnp.float32),
        pltpu.VMEM((G, D), jnp.float32),
    ],
)


def kernel(x, edge_index, batch, W0a, b0a, W0b, b0b, W1a, b1a, W1b, b1b,
           W2a, b2a, W2b, b2b, ln0_g, ln0_b, ln1_g, ln1_b, Wp1, bp1, Wp2, bp2):
    src = edge_index[0].astype(jnp.int32)
    dst = edge_index[1].astype(jnp.int32)
    pad = EP - E
    srcp = jnp.concatenate([src, jnp.zeros((pad,), jnp.int32)])
    dstp = jnp.concatenate([dst, jnp.full((pad,), N, jnp.int32)])
    segs = batch.astype(jnp.int32).reshape(NB, 1, RB)
    zero = jnp.zeros((N, D), jnp.float32)
    pinv = jnp.asarray(_PERM_INV)

    layers = [
        (W0a, b0a, W0b, b0b, ln0_g, ln0_b, True),
        (W1a, b1a, W1b, b1b, ln1_g, ln1_b, True),
        (W2a, b2a, W2b, b2b, ln1_g, ln1_b, False),
    ]
    h = x
    hb = x.astype(jnp.bfloat16)
    for Wa, ba, Wb, bb, g, b, ln in layers:
        hw = lax.bitcast_convert_type(hb.reshape(N, D // 2, 2), jnp.int32)
        a0, a1 = _sc_agg(hw, zero, srcp, dstp)
        Wap = Wa[pinv, :]
        mlp = _mlp_ln if ln else _mlp_plain
        h, hb = mlp(a0, a1, h, Wa, Wap, ba.reshape(1, D), Wb,
                    bb.reshape(1, D), g.reshape(1, D), b.reshape(1, D))
    emb = h
    logp = _pool(emb, segs, Wp1, bp1.reshape(1, D), Wp2, bp2.reshape(1, D))
    return (emb, logp)
